# baseline (speedup 1.0000x reference)
import jax
import jax.numpy as jnp
from jax import lax
from jax.experimental import pallas as pl
from jax.experimental.pallas import tpu as pltpu
from jax.experimental.pallas import tpu_sc as plsc

NUM_COLORS = 100
NUM_SHAPES = 100
D = 128
BATCH = 16384

NC = 2
NS = 16
NW = NC * NS
BPW = BATCH // NW
CHUNK = 128
NCH = BPW // CHUNK
PAIR = 2 * CHUNK
NP = 2 * BPW // PAIR


NRB = 3


def _body(color_tbl, shape_tbl, cidx, sidx, out, idx_v, ctbl_v, stbl_v,
          rows_v, gsem, wsem, isem):
    wid = lax.axis_index("s") * NC + lax.axis_index("c")
    base = wid * BPW

    i0 = pltpu.async_copy(cidx.at[wid], idx_v.at[0], isem.at[0])
    i1 = pltpu.async_copy(sidx.at[wid], idx_v.at[1], isem.at[1])
    @pl.when(lax.axis_index("s") == 0)
    def _stage():
        s0 = pltpu.async_copy(color_tbl, ctbl_v, isem.at[2])
        s1 = pltpu.async_copy(shape_tbl, stbl_v, isem.at[3])
        s0.wait()
        s1.wait()
    i0.wait()
    i1.wait()
    plsc.subcore_barrier()

    sched = [(0, 0, 1), (0, 1, 2), (0, 3, 1), (1, 0, 2), (1, 2, 2)]
    ng = len(sched)
    tbls = (ctbl_v, stbl_v)
    g = [[] for _ in range(ng)]
    w = [None] * ng

    def fire_gathers(p):
        b = p % NRB
        t, j0, n = sched[p]
        for h in range(n):
            g[p].append(pltpu.async_copy(
                tbls[t].at[idx_v.at[t, j0 + h]],
                rows_v.at[b, pl.ds(h * CHUNK, CHUNK)],
                gsem.at[b, h]))

    def fire_write(p):
        b = p % NRB
        t, j0, n = sched[p]
        return pltpu.async_copy(
            rows_v.at[b, pl.ds(0, n * CHUNK)],
            out.at[t, pl.ds(base + j0 * CHUNK, n * CHUNK)],
            wsem.at[b])

    for p in range(ng):
        if p >= NRB:
            w[p - NRB].wait()
        fire_gathers(p)
        if p >= 1:
            for h in g[p - 1]:
                h.wait()
            w[p - 1] = fire_write(p - 1)
    for h in g[ng - 1]:
        h.wait()
    w[ng - 1] = fire_write(ng - 1)
    for p in range(max(0, ng - NRB), ng):
        w[p].wait()


def kernel(color_tbl, shape_tbl, color_idx, shape_idx):
    cidx = color_idx.reshape(NW, NCH, CHUNK)
    sidx = shape_idx.reshape(NW, NCH, CHUNK)
    mesh = plsc.VectorSubcoreMesh(core_axis_name="c", subcore_axis_name="s")
    f = pl.kernel(
        _body,
        out_type=jax.ShapeDtypeStruct((2, BATCH, D), jnp.float32),
        mesh=mesh,
        scratch_types=[
            pltpu.VMEM((2, NCH, CHUNK), jnp.int32),
            pltpu.VMEM_SHARED((NUM_COLORS, D), jnp.float32),
            pltpu.VMEM_SHARED((NUM_SHAPES, D), jnp.float32),
            pltpu.VMEM((NRB, PAIR, D), jnp.float32),
            pltpu.SemaphoreType.DMA((NRB, 2)),
            pltpu.SemaphoreType.DMA((NRB,)),
            pltpu.SemaphoreType.DMA((4,)),
        ],
    )
    return f(color_tbl, shape_tbl, cidx, sidx)


# speedup vs baseline: 1.0093x; 1.0093x over previous
import jax
import jax.numpy as jnp
from jax import lax
from jax.experimental import pallas as pl
from jax.experimental.pallas import tpu as pltpu
from jax.experimental.pallas import tpu_sc as plsc

NUM_COLORS = 100
NUM_SHAPES = 100
D = 128
BATCH = 16384

NC = 2
NS = 16
NW = NC * NS
BPW = BATCH // NW
CHUNK = 128
NCH = BPW // CHUNK
PAIR = 2 * CHUNK
NP = 2 * BPW // PAIR


NRB = 3


def _body(color_tbl, shape_tbl, cidx, sidx, out, idx_v, ctbl_v, stbl_v,
          rows_v, gsem, wsem, isem):
    wid = lax.axis_index("s") * NC + lax.axis_index("c")
    base = wid * BPW

    i0 = pltpu.async_copy(cidx.at[wid], idx_v.at[0], isem.at[0])
    i1 = pltpu.async_copy(sidx.at[wid], idx_v.at[1], isem.at[1])
    @pl.when(lax.axis_index("s") == 0)
    def _stage():
        s0 = pltpu.async_copy(color_tbl, ctbl_v, isem.at[2])
        s1 = pltpu.async_copy(shape_tbl, stbl_v, isem.at[3])
        s0.wait()
        s1.wait()
    i0.wait()
    i1.wait()
    plsc.subcore_barrier()

    tbls = (ctbl_v, ctbl_v, stbl_v, stbl_v)
    g = [None] * (2 * NCH)
    w = [None] * NP

    def fire_gathers(p):
        b = p % NRB
        t, half = p // 2, p % 2
        for h in range(2):
            j = 2 * half + h
            g[2 * p + h] = pltpu.async_copy(
                tbls[p].at[idx_v.at[t, j]],
                rows_v.at[b, pl.ds(h * CHUNK, CHUNK)],
                gsem.at[b, h])

    def fire_write(p):
        b = p % NRB
        t, half = p // 2, p % 2
        return pltpu.async_copy(
            rows_v.at[b],
            out.at[t, pl.ds(base + half * PAIR, PAIR)],
            wsem.at[b])

    for p in range(NP):
        if p >= NRB:
            w[p - NRB].wait()
        fire_gathers(p)
        if p >= 1:
            g[2 * p - 2].wait()
            g[2 * p - 1].wait()
            w[p - 1] = fire_write(p - 1)
    g[2 * NP - 2].wait()
    g[2 * NP - 1].wait()
    w[NP - 1] = fire_write(NP - 1)
    for p in range(max(0, NP - NRB), NP):
        w[p].wait()


def kernel(color_tbl, shape_tbl, color_idx, shape_idx):
    cidx = color_idx.reshape(NW, NCH, CHUNK)
    sidx = shape_idx.reshape(NW, NCH, CHUNK)
    mesh = plsc.VectorSubcoreMesh(core_axis_name="c", subcore_axis_name="s")
    f = pl.kernel(
        _body,
        out_type=jax.ShapeDtypeStruct((2, BATCH, D), jnp.float32),
        mesh=mesh,
        scratch_types=[
            pltpu.VMEM((2, NCH, CHUNK), jnp.int32),
            pltpu.VMEM_SHARED((NUM_COLORS, D), jnp.float32),
            pltpu.VMEM_SHARED((NUM_SHAPES, D), jnp.float32),
            pltpu.VMEM((NRB, PAIR, D), jnp.float32),
            pltpu.SemaphoreType.DMA((NRB, 2)),
            pltpu.SemaphoreType.DMA((NRB,)),
            pltpu.SemaphoreType.DMA((4,)),
        ],
    )
    return f(color_tbl, shape_tbl, cidx, sidx)
